# baseline (speedup 1.0000x reference)
import functools

import jax
import jax.numpy as jnp
import numpy as np
from jax import lax
from jax.experimental import pallas as pl
from jax.experimental.pallas import tpu as pltpu
from jax.experimental.pallas import tpu_sc as plsc

_PI = float(np.arccos(0.0) * 2.0)
_D225 = _PI / 8
_D675 = 3 * _PI / 8
_D1125 = 5 * _PI / 8
_D1575 = 7 * _PI / 8
_D180 = _PI

_W = 512


def _select_nmax(tv, pair_h, pair_d45, pair_v, pair_d135):
    hm = (tv < _D225) | (tv >= _D1575)
    return jnp.where(
        hm, pair_h,
        jnp.where(tv < _D675, pair_d45,
                  jnp.where(tv < _D1125, pair_v, pair_d135)))


_IMGS = 2


def _nms_image(g, t):
    tv = jnp.abs(t)

    left = jnp.concatenate([g[:, :1], g[:, :-1]], axis=1)
    right = jnp.concatenate([g[:, 1:], g[:, -1:]], axis=1)

    def up(x):
        return jnp.concatenate([x[:1], x[:-1]], axis=0)

    def down(x):
        return jnp.concatenate([x[1:], x[-1:]], axis=0)

    pair_h = jnp.maximum(left, right)
    pair_d45 = jnp.maximum(up(right), down(left))
    pair_v = jnp.maximum(up(g), down(g))
    pair_d135 = jnp.maximum(up(left), down(right))

    nmax = _select_nmax(tv, pair_h, pair_d45, pair_v, pair_d135)
    keep = (g >= nmax) & (tv <= _D180)
    return jnp.where(keep, g, jnp.zeros_like(g))


def _tc_body(g_ref, t_ref, o_ref):
    for j in range(_IMGS):
        o_ref[j] = _nms_image(g_ref[j], t_ref[j])


def _tc_nms(g3, t3):
    B = g3.shape[0]
    return pl.pallas_call(
        _tc_body,
        grid=(B // _IMGS,),
        in_specs=[
            pl.BlockSpec((_IMGS, _W, _W), lambda i: (i, 0, 0)),
            pl.BlockSpec((_IMGS, _W, _W), lambda i: (i, 0, 0)),
        ],
        out_specs=pl.BlockSpec((_IMGS, _W, _W), lambda i: (i, 0, 0)),
        out_shape=jax.ShapeDtypeStruct((B, _W, _W), jnp.float32),
        compiler_params=pltpu.CompilerParams(
            dimension_semantics=("parallel",)),
    )(g3, t3)


_NW = 32
_CHUNK = 32


def _sc_chunk_rows(gbuf, tbuf, obuf, mask0, mask15):
    def row_body(y, carry):
        base_up = (y + 1) * _W
        base_me = (y + 2) * _W
        base_dn = (y + 3) * _W
        base_t = y * _W
        for i in range(_CHUNK):
            b = 16 * i
            P = gbuf[pl.ds(base_me + b, 16)]
            up = gbuf[pl.ds(base_up + b, 16)]
            dn = gbuf[pl.ds(base_dn + b, 16)]
            tv = jnp.abs(tbuf[pl.ds(base_t + b, 16)])
            Lup = gbuf[pl.ds(base_up + b - 1, 16)]
            Lme = gbuf[pl.ds(base_me + b - 1, 16)]
            Ldn = gbuf[pl.ds(base_dn + b - 1, 16)]
            if i == 0:
                Lup = jnp.where(mask0, up, Lup)
                Lme = jnp.where(mask0, P, Lme)
                Ldn = jnp.where(mask0, dn, Ldn)
            Rup = gbuf[pl.ds(base_up + b + 1, 16)]
            Rme = gbuf[pl.ds(base_me + b + 1, 16)]
            Rdn = gbuf[pl.ds(base_dn + b + 1, 16)]
            if i == _CHUNK - 1:
                Rup = jnp.where(mask15, up, Rup)
                Rme = jnp.where(mask15, P, Rme)
                Rdn = jnp.where(mask15, dn, Rdn)

            pair_h = jnp.maximum(Lme, Rme)
            pair_v = jnp.maximum(up, dn)
            pair_d45 = jnp.maximum(Rup, Ldn)
            pair_d135 = jnp.maximum(Lup, Rdn)

            nmax = _select_nmax(tv, pair_h, pair_d45, pair_v, pair_d135)
            keep = (P >= nmax) & (tv <= _D180)
            obuf[pl.ds(y * _W + b, 16)] = jnp.where(keep, P, jnp.zeros_like(P))
        return carry

    lax.fori_loop(0, _CHUNK, row_body, 0)


def _make_sc_nms(total_rows):
    rows_per_worker = total_rows // _NW
    n_chunks = rows_per_worker // _CHUNK
    mesh = plsc.VectorSubcoreMesh(core_axis_name="c", subcore_axis_name="s")

    @functools.partial(
        pl.kernel,
        mesh=mesh,
        out_type=jax.ShapeDtypeStruct((total_rows * _W,), jnp.float32),
        scratch_types=[
            pltpu.VMEM(((_CHUNK + 4) * _W,), jnp.float32),
            pltpu.VMEM((_CHUNK * _W,), jnp.float32),
            pltpu.VMEM((_CHUNK * _W,), jnp.float32),
        ],
    )
    def sc_kernel(g_hbm, t_hbm, o_hbm, gbuf, tbuf, obuf):
        c = lax.axis_index("c")
        s = lax.axis_index("s")
        base = (s * 2 + c) * rows_per_worker
        iota = lax.iota(jnp.int32, 16)
        mask0 = iota == 0
        mask15 = iota == 15

        for k in range(n_chunks):
            r0 = base + k * _CHUNK
            y0 = lax.rem(r0, _W)
            up_idx = r0 - (y0 > 0).astype(jnp.int32)
            dn_idx = r0 + _CHUNK - 1 + (y0 + _CHUNK < _W).astype(jnp.int32)

            pltpu.sync_copy(g_hbm.at[pl.ds(r0 * _W, _CHUNK * _W)],
                            gbuf.at[pl.ds(2 * _W, _CHUNK * _W)])
            pltpu.sync_copy(g_hbm.at[pl.ds(up_idx * _W, _W)],
                            gbuf.at[pl.ds(_W, _W)])
            pltpu.sync_copy(g_hbm.at[pl.ds(dn_idx * _W, _W)],
                            gbuf.at[pl.ds((_CHUNK + 2) * _W, _W)])
            pltpu.sync_copy(t_hbm.at[pl.ds(r0 * _W, _CHUNK * _W)], tbuf)

            _sc_chunk_rows(gbuf, tbuf, obuf, mask0, mask15)

            pltpu.sync_copy(obuf, o_hbm.at[pl.ds(r0 * _W, _CHUNK * _W)])

    return sc_kernel


def _sc_nms(g2, t2):
    total_rows = g2.shape[0]
    out = _make_sc_nms(total_rows)(g2.reshape(-1), t2.reshape(-1))
    return out.reshape(total_rows, _W)


def kernel(g, t):
    B, _, H, W = g.shape
    out = _tc_nms(g.reshape(B, H, W), t.reshape(B, H, W))
    return out.reshape(B, 1, H, W)


# speedup vs baseline: 1.0013x; 1.0013x over previous
import jax
import jax.numpy as jnp
import numpy as np
from jax.experimental import pallas as pl
from jax.experimental.pallas import tpu as pltpu

_PI = float(np.arccos(0.0) * 2.0)
_D225 = _PI / 8
_D675 = 3 * _PI / 8
_D1125 = 5 * _PI / 8
_D1575 = 7 * _PI / 8
_D180 = _PI

_W = 512


def _select_nmax(tv, pair_h, pair_d45, pair_v, pair_d135):
    hm = (tv < _D225) | (tv >= _D1575)
    return jnp.where(
        hm, pair_h,
        jnp.where(tv < _D675, pair_d45,
                  jnp.where(tv < _D1125, pair_v, pair_d135)))


_IMGS = 2


def _nms_image(g, t):
    tv = jnp.abs(t)

    left = jnp.concatenate([g[:, :1], g[:, :-1]], axis=1)
    right = jnp.concatenate([g[:, 1:], g[:, -1:]], axis=1)

    def up(x):
        return jnp.concatenate([x[:1], x[:-1]], axis=0)

    def down(x):
        return jnp.concatenate([x[1:], x[-1:]], axis=0)

    pair_h = jnp.maximum(left, right)
    pair_d45 = jnp.maximum(up(right), down(left))
    pair_v = jnp.maximum(up(g), down(g))
    pair_d135 = jnp.maximum(up(left), down(right))

    nmax = _select_nmax(tv, pair_h, pair_d45, pair_v, pair_d135)
    keep = (g >= nmax) & (tv <= _D180)
    return jnp.where(keep, g, jnp.zeros_like(g))


def _tc_body(g_ref, t_ref, o_ref):
    for j in range(_IMGS):
        o_ref[j] = _nms_image(g_ref[j], t_ref[j])


def _tc_nms(g3, t3):
    B = g3.shape[0]
    return pl.pallas_call(
        _tc_body,
        grid=(B // _IMGS,),
        in_specs=[
            pl.BlockSpec((_IMGS, _W, _W), lambda i: (i, 0, 0)),
            pl.BlockSpec((_IMGS, _W, _W), lambda i: (i, 0, 0)),
        ],
        out_specs=pl.BlockSpec((_IMGS, _W, _W), lambda i: (i, 0, 0)),
        out_shape=jax.ShapeDtypeStruct((B, _W, _W), jnp.float32),
        compiler_params=pltpu.CompilerParams(
            dimension_semantics=("parallel",)),
    )(g3, t3)


def kernel(g, t):
    B, _, H, W = g.shape
    out = _tc_nms(g.reshape(B, H, W), t.reshape(B, H, W))
    return out.reshape(B, 1, H, W)


# speedup vs baseline: 1.0090x; 1.0077x over previous
import jax
import jax.numpy as jnp
import numpy as np
from jax.experimental import pallas as pl
from jax.experimental.pallas import tpu as pltpu

_PI = float(np.arccos(0.0) * 2.0)
_D225 = _PI / 8
_D675 = 3 * _PI / 8
_D1125 = 5 * _PI / 8
_D1575 = 7 * _PI / 8
_D180 = _PI

_W = 512


def _select_nmax(tv, pair_h, pair_d45, pair_v, pair_d135):
    hm = (tv < _D225) | (tv >= _D1575)
    return jnp.where(
        hm, pair_h,
        jnp.where(tv < _D675, pair_d45,
                  jnp.where(tv < _D1125, pair_v, pair_d135)))


_IMGS = 2


def _nms_image(g, t):
    tv = jnp.abs(t)

    left = jnp.concatenate([g[:, :, :1], g[:, :, :-1]], axis=2)
    right = jnp.concatenate([g[:, :, 1:], g[:, :, -1:]], axis=2)

    def up(x):
        return jnp.concatenate([x[:, :1], x[:, :-1]], axis=1)

    def down(x):
        return jnp.concatenate([x[:, 1:], x[:, -1:]], axis=1)

    pair_h = jnp.maximum(left, right)
    pair_d45 = jnp.maximum(up(right), down(left))
    pair_v = jnp.maximum(up(g), down(g))
    pair_d135 = jnp.maximum(up(left), down(right))

    nmax = _select_nmax(tv, pair_h, pair_d45, pair_v, pair_d135)
    keep = (g >= nmax) & (tv <= _D180)
    return jnp.where(keep, g, jnp.zeros_like(g))


def _tc_body(g_ref, t_ref, o_ref):
    o_ref[...] = _nms_image(g_ref[...], t_ref[...])


def _tc_nms(g3, t3):
    B = g3.shape[0]
    return pl.pallas_call(
        _tc_body,
        grid=(B // _IMGS,),
        in_specs=[
            pl.BlockSpec((_IMGS, _W, _W), lambda i: (i, 0, 0)),
            pl.BlockSpec((_IMGS, _W, _W), lambda i: (i, 0, 0)),
        ],
        out_specs=pl.BlockSpec((_IMGS, _W, _W), lambda i: (i, 0, 0)),
        out_shape=jax.ShapeDtypeStruct((B, _W, _W), jnp.float32),
        compiler_params=pltpu.CompilerParams(
            dimension_semantics=("parallel",)),
    )(g3, t3)


def kernel(g, t):
    B, _, H, W = g.shape
    out = _tc_nms(g.reshape(B, H, W), t.reshape(B, H, W))
    return out.reshape(B, 1, H, W)
